# baseline (speedup 1.0000x reference)
import jax
import jax.numpy as jnp
from jax.experimental import pallas as pl


def _copy_row(x_ref, o_ref):
    o_ref[...] = x_ref[0:1, :]


def kernel(x):
    return pl.pallas_call(
        _copy_row,
        grid=(1,),
        out_shape=jax.ShapeDtypeStruct((1, 128), jnp.float32),
        in_specs=[pl.BlockSpec((8, 128), lambda i: (0, 0))],
        out_specs=pl.BlockSpec((1, 128), lambda i: (0, 0)),
    )(x)
